# Initial kernel scaffold; baseline (speedup 1.0000x reference)
import jax
import jax.numpy as jnp
from jax.experimental import pallas as pl


def kernel(feature_matrix, G, adj_out, adj_in, c1, c2, hW1, hb1, hW2, hb2, g1W1, g1b1, g1W2, g1b2, g2W1, g2b1, g2W2, g2b2, w1W, w1b, w2W, w2b, w3W, w3b, w4W, w4b, s1fW, s1fb, s1oW, s1ob, s2fW, s2fb, s2oW, s2ob, s3fW, s3fb, s3oW, s3ob):
    raise NotImplementedError("write your pallas kernel here")


# speedup vs baseline: 1.2332x; 1.2332x over previous
import jax
import jax.numpy as jnp
from jax.experimental import pallas as pl
from jax.experimental.pallas import tpu as pltpu
from jax.experimental.pallas import tpu_sc as plsc

N = 4096
IN_CH = 128
HID = 64
OUT = 64
B = 4096

BM = 256
NB = N // BM
GATHER_W = 128

f32 = jnp.float32
bf16 = jnp.bfloat16


def _gcn_two_pass(a, fm, w1, pre1b, w2, midb, pre2b, postb):
    def body(a_ref, fm_ref, w1_ref, pre1b_ref, w2_ref, midb_ref, pre2b_ref,
             postb_ref, out_ref, abf_scr, y1_scr, x_scr, y2_scr):
        p = pl.program_id(0)
        i = pl.program_id(1)

        @pl.when(jnp.logical_and(p == 0, i == 0))
        def _():
            y1_scr[...] = jnp.dot(fm_ref[...], w1_ref[...],
                                  preferred_element_type=f32) + pre1b_ref[...]

        @pl.when(p == 0)
        def _():
            a = a_ref[...]
            abf_scr[pl.ds(i * BM, BM), :] = a.astype(bf16)
            p1 = jnp.dot(a, y1_scr[...], preferred_element_type=f32)
            x_scr[pl.ds(i * BM, BM), :] = jnp.maximum(p1 + midb_ref[...], 0.0)

        @pl.when(jnp.logical_and(p == 1, i == 0))
        def _():
            y2_scr[...] = jnp.dot(x_scr[...], w2_ref[...],
                                  preferred_element_type=f32) + pre2b_ref[...]

        @pl.when(p == 1)
        def _():
            af = abf_scr[pl.ds(i * BM, BM), :].astype(f32)
            out_ref[...] = jnp.dot(af, y2_scr[...],
                                   preferred_element_type=f32) + postb_ref[...]

    return pl.pallas_call(
        body,
        grid=(2, NB),
        in_specs=[
            pl.BlockSpec((BM, N), lambda p, i: (jnp.where(p == 0, i, NB - 1), 0)),
            pl.BlockSpec((N, IN_CH), lambda p, i: (0, 0)),
            pl.BlockSpec((IN_CH, HID), lambda p, i: (0, 0)),
            pl.BlockSpec((1, HID), lambda p, i: (0, 0)),
            pl.BlockSpec((HID, OUT), lambda p, i: (0, 0)),
            pl.BlockSpec((1, HID), lambda p, i: (0, 0)),
            pl.BlockSpec((1, OUT), lambda p, i: (0, 0)),
            pl.BlockSpec((1, OUT), lambda p, i: (0, 0)),
        ],
        out_specs=pl.BlockSpec((BM, OUT), lambda p, i: (i, 0)),
        out_shape=jax.ShapeDtypeStruct((N, OUT), f32),
        scratch_shapes=[
            pltpu.VMEM((N, N), bf16),
            pltpu.VMEM((N, HID), f32),
            pltpu.VMEM((N, HID), f32),
            pltpu.VMEM((N, OUT), f32),
        ],
    )(a, fm, w1, pre1b, w2, midb, pre2b, postb)


def _epilogue(xh, xo, xi, w1W, w1b, w2W, w2b, w3W, w3b, w4W, w4b,
              s1fW, s1fb, s2fW, s2fb, s3fW, s3fb):
    def body(xh_ref, xo_ref, xi_ref, w1W_ref, w1b_ref, w2W_ref, w2b_ref,
             w3W_ref, w3b_ref, w4W_ref, w4b_ref, s1fW_ref, s1fb_ref,
             s2fW_ref, s2fb_ref, s3fW_ref, s3fb_ref, out_ref):
        xh = xh_ref[...]
        xo = xo_ref[...]
        xi = xi_ref[...]

        def dot(a, b):
            return jnp.dot(a, b, preferred_element_type=f32)

        xn = jnp.maximum(
            (dot(xo, w1W_ref[...]) + w1b_ref[...])
            + (dot(xi, w2W_ref[...]) + w2b_ref[...]), 0.0)
        th = jax.nn.sigmoid(
            (dot(xh, w3W_ref[...]) + w3b_ref[...])
            + (dot(xn, w4W_ref[...]) + w4b_ref[...]))
        xt = th * xh + (1.0 - th) * xn
        z = jnp.zeros((N, 128 - OUT), f32)
        h1 = jnp.maximum(dot(xh, s1fW_ref[...]) + s1fb_ref[...], 0.0)
        h2 = jnp.maximum(dot(xn, s2fW_ref[...]) + s2fb_ref[...], 0.0)
        h3 = jnp.maximum(dot(xt, s3fW_ref[...]) + s3fb_ref[...], 0.0)
        out_ref[0:N, :] = jnp.concatenate([h1, z], axis=1)
        out_ref[N:2 * N, :] = jnp.concatenate([h2, z], axis=1)
        out_ref[2 * N:3 * N, :] = jnp.concatenate([h3, z], axis=1)

    return pl.pallas_call(
        body,
        out_shape=jax.ShapeDtypeStruct((3 * N, 128), f32),
    )(xh, xo, xi, w1W, w1b, w2W, w2b, w3W, w3b, w4W, w4b,
      s1fW, s1fb, s2fW, s2fb, s3fW, s3fb)


def _sc_gather(hcat, idx):
    num_idx = idx.shape[0]
    idx2 = idx.reshape(1, num_idx)
    mesh = plsc.VectorSubcoreMesh(core_axis_name="core", subcore_axis_name="subcore")

    @pl.kernel(out_type=jax.ShapeDtypeStruct((num_idx, 128), f32),
               mesh=mesh)
    def k(h_hbm, i_hbm, o_hbm):
        def body(i_vmem, o_vmem):
            pltpu.sync_copy(h_hbm.at[i_vmem.at[0]], o_vmem)

        pltpu.emit_pipeline(
            body,
            grid=(num_idx // GATHER_W,),
            in_specs=[pl.BlockSpec((1, GATHER_W), lambda i: (0, i))],
            out_specs=[pl.BlockSpec((GATHER_W, 128), lambda i: (i, 0))],
            core_axis_name=("core", "subcore"),
            dimension_semantics=(pltpu.PARALLEL,),
        )(i_hbm, o_hbm)

    return k(hcat, idx2)


def _heads(g, oW1, ob1, oW2, ob2, oW3, ob3):
    def body(g_ref, oW1_ref, ob1_ref, oW2_ref, ob2_ref, oW3_ref, ob3_ref,
             o1_ref, o2_ref, o3_ref):
        for h, (oW_ref, ob_ref, o_ref) in enumerate(
                ((oW1_ref, ob1_ref, o1_ref), (oW2_ref, ob2_ref, o2_ref),
                 (oW3_ref, ob3_ref, o3_ref))):
            a = g_ref[2 * h * B:(2 * h + 1) * B, :]
            b = g_ref[(2 * h + 1) * B:(2 * h + 2) * B, :]
            d = jnp.abs(a - b).astype(bf16).astype(f32)
            w = oW_ref[...].astype(bf16).astype(f32)
            o_ref[...] = (jnp.sum(d * w, axis=1, keepdims=True)
                          + ob_ref[...])

    return pl.pallas_call(
        body,
        out_shape=(jax.ShapeDtypeStruct((B, 1), f32),
                   jax.ShapeDtypeStruct((B, 1), f32),
                   jax.ShapeDtypeStruct((B, 1), f32)),
    )(g, oW1, ob1, oW2, ob2, oW3, ob3)


def kernel(feature_matrix, G, adj_out, adj_in, c1, c2, hW1, hb1, hW2, hb2,
           g1W1, g1b1, g1W2, g1b2, g2W1, g2b1, g2W2, g2b2, w1W, w1b, w2W,
           w2b, w3W, w3b, w4W, w4b, s1fW, s1fb, s1oW, s1ob, s2fW, s2fb,
           s2oW, s2ob, s3fW, s3fb, s3oW, s3ob):
    zeros_h = jnp.zeros((1, HID), f32)
    zeros_o = jnp.zeros((1, OUT), f32)

    x_H = _gcn_two_pass(G, feature_matrix, hW1, hb1.reshape(1, HID),
                        hW2, zeros_h, hb2.reshape(1, OUT), zeros_o)
    x_out = _gcn_two_pass(adj_out, feature_matrix, g1W1, zeros_h,
                          g1W2, g1b1.reshape(1, HID), zeros_o,
                          g1b2.reshape(1, OUT))
    x_in = _gcn_two_pass(adj_in, feature_matrix, g2W1, zeros_h,
                         g2W2, g2b1.reshape(1, HID), zeros_o,
                         g2b2.reshape(1, OUT))

    hcat = _epilogue(x_H, x_out, x_in, w1W, w1b.reshape(1, OUT), w2W,
                     w2b.reshape(1, OUT), w3W, w3b.reshape(1, OUT), w4W,
                     w4b.reshape(1, OUT), s1fW, s1fb.reshape(1, OUT),
                     s2fW, s2fb.reshape(1, OUT), s3fW, s3fb.reshape(1, OUT))

    c1i = c1.astype(jnp.int32)
    c2i = c2.astype(jnp.int32)
    idx = jnp.concatenate([c1i, c2i, c1i + N, c2i + N, c1i + 2 * N, c2i + 2 * N])
    gathered = _sc_gather(hcat, idx)

    def _pad_oW(w):
        return jnp.concatenate(
            [w[:, 0], jnp.zeros((128 - OUT,), f32)]).reshape(1, 128)

    logit_H, logit_N, logit_T = _heads(
        gathered, _pad_oW(s1oW), s1ob.reshape(1, 1),
        _pad_oW(s2oW), s2ob.reshape(1, 1),
        _pad_oW(s3oW), s3ob.reshape(1, 1))
    return (logit_H, logit_N, logit_T)
